# Initial kernel scaffold; baseline (speedup 1.0000x reference)
import jax
import jax.numpy as jnp
from jax.experimental import pallas as pl


def kernel(x, norm_alpha, norm_weight, norm_bias, W_in, b_in, W_out, b_out, A, B_param, C_param, conv_w, W_q, b_q, W_k, b_k):
    raise NotImplementedError("write your pallas kernel here")


# speedup vs baseline: 3.8676x; 3.8676x over previous
import functools

import jax
import jax.numpy as jnp
from jax.experimental import pallas as pl

F32 = jnp.float32


def _k1_body(x_ref, win_ref, bin_ref, wq_ref, bq_ref, wk_ref, bk_ref,
             alpha_ref, nw_ref, nb_ref, xpn_ref, q_ref, k_ref):
    x = x_ref[...]
    xn = jnp.tanh(alpha_ref[0, 0] * x) * nw_ref[...] + nb_ref[...]
    xp = jnp.dot(xn, win_ref[...], preferred_element_type=F32) + bin_ref[...]
    ss = jnp.sum(xp * xp, axis=1, keepdims=True)
    nrm = jnp.maximum(jnp.sqrt(ss), 1e-12)
    xpn = xp / nrm
    xpn_ref[...] = xpn
    q_ref[...] = jnp.dot(xpn, wq_ref[...], preferred_element_type=F32) + bq_ref[0, 0]
    k_ref[...] = jnp.dot(xpn, wk_ref[...], preferred_element_type=F32) + bk_ref[0, 0]


def _k2a_body(qcol_ref, krow_ref, imp_ref, *, L, CH):
    krow = krow_ref[0]
    acc = jnp.zeros((1, L), F32)
    for c in range(L // CH):
        qc = qcol_ref[0, c * CH:(c + 1) * CH, :]
        s = qc * krow
        m = jnp.max(s, axis=1, keepdims=True)
        e = jnp.exp(s - m)
        z = jnp.sum(e, axis=1, keepdims=True)
        acc = acc + jnp.sum(e / z, axis=0, keepdims=True)
    imp_ref[0] = acc * (1.0 / L)


def _k2b_body(imp_row_ref, imp_col_ref, rank_ref, *, L, CH):
    row = imp_row_ref[0]
    for c in range(L // CH):
        col = imp_col_ref[0, c * CH:(c + 1) * CH, :]
        jidx = jax.lax.broadcasted_iota(jnp.int32, (CH, L), 1)
        iidx = jax.lax.broadcasted_iota(jnp.int32, (CH, L), 0) + c * CH
        gt = (row > col).astype(jnp.int32)
        eq = ((row == col) & (jidx < iidx)).astype(jnp.int32)
        rank = jnp.sum(gt + eq, axis=1, keepdims=True)
        rank_ref[0, c * CH:(c + 1) * CH, :] = rank


def _k3_body(rank_ref, xpn_ref, wts_ref, xc_ref, *, KP, K, E):
    rank = rank_ref[0]
    r_iota = jax.lax.broadcasted_iota(jnp.int32, (KP, rank.shape[1]), 0)
    p = jnp.where((rank == r_iota) & (r_iota < K), 1.0, 0.0).astype(F32)
    sf = jnp.dot(p, xpn_ref[...], preferred_element_type=F32)
    acc = wts_ref[3:4, :] * sf
    for j in range(3):
        shift = 3 - j
        sh = jnp.concatenate(
            [jnp.zeros((shift, E), F32), sf[:KP - shift, :]], axis=0)
        acc = acc + wts_ref[j:j + 1, :] * sh
    xc_ref[...] = acc


def _k4_body(xc_ref, g_ref, vt2_ref, mr_ref, apow_ref, out_ref, *, KP, TC, E, S):
    h = jnp.zeros((S, E), F32)
    g = g_ref[...]
    vt2 = vt2_ref[...]
    mr = mr_ref[...]
    apow = apow_ref[...]
    for c in range(KP // TC):
        xc = xc_ref[c * TC:(c + 1) * TC, :]
        carry = jnp.sum(vt2 * h[None, :, :], axis=1)
        xp = jnp.concatenate([jnp.zeros((TC - 1, E), F32), xc], axis=0)
        intra = g[0:1, :] * xc
        for d in range(1, TC):
            intra = intra + g[d:d + 1, :] * xp[TC - 1 - d:2 * TC - 1 - d, :]
        out_ref[c * TC:(c + 1) * TC, :] = intra + carry
        h = (jnp.dot(apow, h, preferred_element_type=F32)
             + jnp.dot(mr, xc, preferred_element_type=F32))


def _k5_body(xssm_ref, wout_ref, bout_ref, xproc_ref):
    xproc_ref[...] = (jnp.dot(xssm_ref[...], wout_ref[...],
                              preferred_element_type=F32) + bout_ref[...])


def _k6_body(rank_ref, xproc_ref, x_ref, out_ref, *, KP, K):
    rcol = rank_ref[0]
    r_iota = jax.lax.broadcasted_iota(jnp.int32, (rcol.shape[0], KP), 1)
    pt = jnp.where((rcol == r_iota) & (r_iota < K), 1.0, 0.0).astype(F32)
    out_ref[...] = (jnp.dot(pt, xproc_ref[...], preferred_element_type=F32)
                    + x_ref[...])


def _ssm_tables(A, B_param, C_param, TC):
    S = A.shape[0]
    bsig = jax.nn.sigmoid(B_param).reshape(1, S)
    csigT = jax.nn.sigmoid(C_param).T
    AT = A.T
    U = csigT[None]
    m = bsig
    P = AT
    n = 1
    while n < TC:
        U = jnp.concatenate([U, jnp.einsum('sk,dkm->dsm', P, U)], axis=0)
        m = jnp.concatenate([m, m @ P], axis=0)
        P = P @ P
        n *= 2
    U = U[:TC]
    m = m[:TC]
    G = jnp.einsum('os,dsm->dm', bsig, U)
    VT2 = jnp.einsum('sk,dkm->dsm', AT, U)
    MR = m[::-1].T
    A_tc = P.T
    return G, VT2, MR, A_tc


def kernel(x, norm_alpha, norm_weight, norm_bias, W_in, b_in, W_out, b_out,
           A, B_param, C_param, conv_w, W_q, b_q, W_k, b_k):
    B, L, C = x.shape
    E = W_in.shape[1]
    S = A.shape[0]
    K = max(1, int(L * 0.3))
    TC = 64
    KP = ((K + TC - 1) // TC) * TC
    RT = 256
    CH = 256
    N = B * L

    xf = x.reshape(N, C)
    alpha2 = norm_alpha.reshape(1, 1)
    nw2 = norm_weight.reshape(1, C)
    nb2 = norm_bias.reshape(1, C)
    bin2 = b_in.reshape(1, E)
    bout2 = b_out.reshape(1, C)
    bq2 = b_q.reshape(1, 1)
    bk2 = b_k.reshape(1, 1)
    wts = jnp.concatenate([conv_w[:, 0, :].T, jnp.zeros((4, E), F32)], axis=0)

    xpn, q, k = pl.pallas_call(
        _k1_body,
        grid=(N // RT,),
        in_specs=[
            pl.BlockSpec((RT, C), lambda i: (i, 0)),
            pl.BlockSpec((C, E), lambda i: (0, 0)),
            pl.BlockSpec((1, E), lambda i: (0, 0)),
            pl.BlockSpec((E, 1), lambda i: (0, 0)),
            pl.BlockSpec((1, 1), lambda i: (0, 0)),
            pl.BlockSpec((E, 1), lambda i: (0, 0)),
            pl.BlockSpec((1, 1), lambda i: (0, 0)),
            pl.BlockSpec((1, 1), lambda i: (0, 0)),
            pl.BlockSpec((1, C), lambda i: (0, 0)),
            pl.BlockSpec((1, C), lambda i: (0, 0)),
        ],
        out_specs=[
            pl.BlockSpec((RT, E), lambda i: (i, 0)),
            pl.BlockSpec((RT, 1), lambda i: (i, 0)),
            pl.BlockSpec((RT, 1), lambda i: (i, 0)),
        ],
        out_shape=[
            jax.ShapeDtypeStruct((N, E), F32),
            jax.ShapeDtypeStruct((N, 1), F32),
            jax.ShapeDtypeStruct((N, 1), F32),
        ],
    )(xf, W_in, bin2, W_q, bq2, W_k, bk2, alpha2, nw2, nb2)

    qcol = q.reshape(B, L, 1)
    krow = k.reshape(B, 1, L)

    imp = pl.pallas_call(
        functools.partial(_k2a_body, L=L, CH=CH),
        grid=(B,),
        in_specs=[
            pl.BlockSpec((1, L, 1), lambda b: (b, 0, 0)),
            pl.BlockSpec((1, 1, L), lambda b: (b, 0, 0)),
        ],
        out_specs=pl.BlockSpec((1, 1, L), lambda b: (b, 0, 0)),
        out_shape=jax.ShapeDtypeStruct((B, 1, L), F32),
    )(qcol, krow)

    imp_col = jnp.swapaxes(imp, 1, 2)

    rank = pl.pallas_call(
        functools.partial(_k2b_body, L=L, CH=CH),
        grid=(B,),
        in_specs=[
            pl.BlockSpec((1, 1, L), lambda b: (b, 0, 0)),
            pl.BlockSpec((1, L, 1), lambda b: (b, 0, 0)),
        ],
        out_specs=pl.BlockSpec((1, L, 1), lambda b: (b, 0, 0)),
        out_shape=jax.ShapeDtypeStruct((B, L, 1), jnp.int32),
    )(imp, imp_col)

    rank_row = jnp.swapaxes(rank, 1, 2)

    xc = pl.pallas_call(
        functools.partial(_k3_body, KP=KP, K=K, E=E),
        grid=(B,),
        in_specs=[
            pl.BlockSpec((1, 1, L), lambda b: (b, 0, 0)),
            pl.BlockSpec((L, E), lambda b: (b, 0)),
            pl.BlockSpec((8, E), lambda b: (0, 0)),
        ],
        out_specs=pl.BlockSpec((KP, E), lambda b: (b, 0)),
        out_shape=jax.ShapeDtypeStruct((B * KP, E), F32),
    )(rank_row, xpn, wts)

    G, VT2, MR, A_tc = _ssm_tables(A, B_param, C_param, TC)
    xssm = pl.pallas_call(
        functools.partial(_k4_body, KP=KP, TC=TC, E=E, S=S),
        grid=(B,),
        in_specs=[
            pl.BlockSpec((KP, E), lambda b: (b, 0)),
            pl.BlockSpec((TC, E), lambda b: (0, 0)),
            pl.BlockSpec((TC, S, E), lambda b: (0, 0, 0)),
            pl.BlockSpec((S, TC), lambda b: (0, 0)),
            pl.BlockSpec((S, S), lambda b: (0, 0)),
        ],
        out_specs=pl.BlockSpec((KP, E), lambda b: (b, 0)),
        out_shape=jax.ShapeDtypeStruct((B * KP, E), F32),
    )(xc, G, VT2, MR, A_tc)

    xproc = pl.pallas_call(
        _k5_body,
        grid=(B,),
        in_specs=[
            pl.BlockSpec((KP, E), lambda b: (b, 0)),
            pl.BlockSpec((E, C), lambda b: (0, 0)),
            pl.BlockSpec((1, C), lambda b: (0, 0)),
        ],
        out_specs=pl.BlockSpec((KP, C), lambda b: (b, 0)),
        out_shape=jax.ShapeDtypeStruct((B * KP, C), F32),
    )(xssm, W_out, bout2)

    TLS = L // 8
    out = pl.pallas_call(
        functools.partial(_k6_body, KP=KP, K=K),
        grid=(B, 8),
        in_specs=[
            pl.BlockSpec((1, TLS, 1), lambda b, t: (b, t, 0)),
            pl.BlockSpec((KP, C), lambda b, t: (b, 0)),
            pl.BlockSpec((TLS, C), lambda b, t: (b * 8 + t, 0)),
        ],
        out_specs=pl.BlockSpec((TLS, C), lambda b, t: (b * 8 + t, 0)),
        out_shape=jax.ShapeDtypeStruct((N, C), F32),
    )(rank, xproc, xf)

    return out.reshape(B, L, C)


# speedup vs baseline: 4.5998x; 1.1893x over previous
import functools

import jax
import jax.numpy as jnp
from jax.experimental import pallas as pl

F32 = jnp.float32
BF16 = jnp.bfloat16


def _k1_body(x_ref, win_ref, bin_ref, wq_ref, bq_ref, wk_ref, bk_ref,
             alpha_ref, nw_ref, nb_ref, xpn_ref, q_ref, k_ref):
    x = x_ref[...]
    xn = jnp.tanh(alpha_ref[0, 0] * x) * nw_ref[...] + nb_ref[...]
    xp = jnp.dot(xn.astype(BF16), win_ref[...],
                 preferred_element_type=F32) + bin_ref[...]
    ss = jnp.sum(xp * xp, axis=1, keepdims=True)
    nrm = jnp.maximum(jnp.sqrt(ss), 1e-12)
    xpn = xp / nrm
    xpn_ref[...] = xpn.astype(BF16)
    q_ref[...] = jnp.dot(xpn, wq_ref[...], preferred_element_type=F32) + bq_ref[0, 0]
    k_ref[...] = jnp.dot(xpn, wk_ref[...], preferred_element_type=F32) + bk_ref[0, 0]


def _k2a_body(qcol_ref, krow_ref, imp_ref, *, L, CH):
    krow = krow_ref[0]
    acc = jnp.zeros((1, L), F32)
    for c in range(L // CH):
        qc = qcol_ref[0, c * CH:(c + 1) * CH, :]
        s = qc * krow
        m = jnp.max(s, axis=1, keepdims=True)
        e = jnp.exp(s - m)
        z = jnp.sum(e, axis=1, keepdims=True)
        acc = acc + jnp.sum(e / z, axis=0, keepdims=True)
    imp_ref[0] = acc * (1.0 / L)


def _k2b_body(imp_row_ref, imp_col_ref, rank_ref, *, L, CH):
    row = imp_row_ref[0]
    for c in range(L // CH):
        col = imp_col_ref[0, c * CH:(c + 1) * CH, :]
        jidx = jax.lax.broadcasted_iota(jnp.int32, (CH, L), 1)
        iidx = jax.lax.broadcasted_iota(jnp.int32, (CH, L), 0) + c * CH
        gt = (row > col).astype(jnp.int32)
        eq = ((row == col) & (jidx < iidx)).astype(jnp.int32)
        rank = jnp.sum(gt + eq, axis=1, keepdims=True)
        rank_ref[0, c * CH:(c + 1) * CH, :] = rank


def _k3_body(rank_ref, xpn_ref, wts_ref, xc_ref, *, KP, K, E):
    rank = rank_ref[0]
    r_iota = jax.lax.broadcasted_iota(jnp.int32, (KP, rank.shape[1]), 0)
    p = jnp.where((rank == r_iota) & (r_iota < K), 1.0, 0.0).astype(BF16)
    sf = jnp.dot(p, xpn_ref[...], preferred_element_type=F32)
    acc = wts_ref[3:4, :] * sf
    for j in range(3):
        shift = 3 - j
        sh = jnp.concatenate(
            [jnp.zeros((shift, E), F32), sf[:KP - shift, :]], axis=0)
        acc = acc + wts_ref[j:j + 1, :] * sh
    xc_ref[...] = acc


def _k45_body(xc_ref, g_ref, vt2_ref, mr_ref, apow_ref, wout_ref, bout_ref,
              xproc_ref, *, KP, TC, E, S):
    h = jnp.zeros((S, E), F32)
    g = g_ref[...]
    vt2 = vt2_ref[...]
    mr = mr_ref[...]
    apow = apow_ref[...]
    wout = wout_ref[...]
    for c in range(KP // TC):
        xc = xc_ref[c * TC:(c + 1) * TC, :]
        carry = jnp.sum(vt2 * h[None, :, :], axis=1)
        xp = jnp.concatenate([jnp.zeros((TC - 1, E), F32), xc], axis=0)
        intra = g[0:1, :] * xc
        for d in range(1, TC):
            intra = intra + g[d:d + 1, :] * xp[TC - 1 - d:2 * TC - 1 - d, :]
        xssm = intra + carry
        xproc_ref[c * TC:(c + 1) * TC, :] = (
            jnp.dot(xssm.astype(BF16), wout, preferred_element_type=F32)
            + bout_ref[...]).astype(BF16)
        h = (jnp.dot(apow, h, preferred_element_type=F32)
             + jnp.dot(mr, xc, preferred_element_type=F32))


def _k6_body(rank_ref, xproc_ref, x_ref, out_ref, *, KP, K):
    rcol = rank_ref[0]
    r_iota = jax.lax.broadcasted_iota(jnp.int32, (rcol.shape[0], KP), 1)
    pt = jnp.where((rcol == r_iota) & (r_iota < K), 1.0, 0.0).astype(BF16)
    out_ref[...] = (jnp.dot(pt, xproc_ref[...], preferred_element_type=F32)
                    + x_ref[...])


def _ssm_tables(A, B_param, C_param, TC):
    S = A.shape[0]
    bsig = jax.nn.sigmoid(B_param).reshape(1, S)
    csigT = jax.nn.sigmoid(C_param).T
    AT = A.T
    U = csigT[None]
    m = bsig
    P = AT
    n = 1
    while n < TC:
        U = jnp.concatenate([U, jnp.einsum('sk,dkm->dsm', P, U)], axis=0)
        m = jnp.concatenate([m, m @ P], axis=0)
        P = P @ P
        n *= 2
    U = U[:TC]
    m = m[:TC]
    G = jnp.einsum('os,dsm->dm', bsig, U)
    VT2 = jnp.einsum('sk,dkm->dsm', AT, U)
    MR = m[::-1].T
    A_tc = P.T
    return G, VT2, MR, A_tc


def kernel(x, norm_alpha, norm_weight, norm_bias, W_in, b_in, W_out, b_out,
           A, B_param, C_param, conv_w, W_q, b_q, W_k, b_k):
    B, L, C = x.shape
    E = W_in.shape[1]
    S = A.shape[0]
    K = max(1, int(L * 0.3))
    TC = 64
    KP = ((K + TC - 1) // TC) * TC
    RT = 256
    CH = 256
    N = B * L

    xf = x.reshape(N, C)
    alpha2 = norm_alpha.reshape(1, 1)
    nw2 = norm_weight.reshape(1, C)
    nb2 = norm_bias.reshape(1, C)
    bin2 = b_in.reshape(1, E)
    bout2 = b_out.reshape(1, C)
    bq2 = b_q.reshape(1, 1)
    bk2 = b_k.reshape(1, 1)
    wts = jnp.concatenate([conv_w[:, 0, :].T, jnp.zeros((4, E), F32)], axis=0)

    xpn, q, k = pl.pallas_call(
        _k1_body,
        grid=(N // RT,),
        in_specs=[
            pl.BlockSpec((RT, C), lambda i: (i, 0)),
            pl.BlockSpec((C, E), lambda i: (0, 0)),
            pl.BlockSpec((1, E), lambda i: (0, 0)),
            pl.BlockSpec((E, 1), lambda i: (0, 0)),
            pl.BlockSpec((1, 1), lambda i: (0, 0)),
            pl.BlockSpec((E, 1), lambda i: (0, 0)),
            pl.BlockSpec((1, 1), lambda i: (0, 0)),
            pl.BlockSpec((1, 1), lambda i: (0, 0)),
            pl.BlockSpec((1, C), lambda i: (0, 0)),
            pl.BlockSpec((1, C), lambda i: (0, 0)),
        ],
        out_specs=[
            pl.BlockSpec((RT, E), lambda i: (i, 0)),
            pl.BlockSpec((RT, 1), lambda i: (i, 0)),
            pl.BlockSpec((RT, 1), lambda i: (i, 0)),
        ],
        out_shape=[
            jax.ShapeDtypeStruct((N, E), BF16),
            jax.ShapeDtypeStruct((N, 1), F32),
            jax.ShapeDtypeStruct((N, 1), F32),
        ],
    )(xf, W_in.astype(BF16), bin2, W_q, bq2, W_k, bk2, alpha2, nw2, nb2)

    qcol = q.reshape(B, L, 1)
    krow = k.reshape(B, 1, L)

    imp = pl.pallas_call(
        functools.partial(_k2a_body, L=L, CH=CH),
        grid=(B,),
        in_specs=[
            pl.BlockSpec((1, L, 1), lambda b: (b, 0, 0)),
            pl.BlockSpec((1, 1, L), lambda b: (b, 0, 0)),
        ],
        out_specs=pl.BlockSpec((1, 1, L), lambda b: (b, 0, 0)),
        out_shape=jax.ShapeDtypeStruct((B, 1, L), F32),
    )(qcol, krow)

    imp_col = jnp.swapaxes(imp, 1, 2)

    rank = pl.pallas_call(
        functools.partial(_k2b_body, L=L, CH=CH),
        grid=(B,),
        in_specs=[
            pl.BlockSpec((1, 1, L), lambda b: (b, 0, 0)),
            pl.BlockSpec((1, L, 1), lambda b: (b, 0, 0)),
        ],
        out_specs=pl.BlockSpec((1, L, 1), lambda b: (b, 0, 0)),
        out_shape=jax.ShapeDtypeStruct((B, L, 1), jnp.int32),
    )(imp, imp_col)

    rank_row = jnp.swapaxes(rank, 1, 2)

    xc = pl.pallas_call(
        functools.partial(_k3_body, KP=KP, K=K, E=E),
        grid=(B,),
        in_specs=[
            pl.BlockSpec((1, 1, L), lambda b: (b, 0, 0)),
            pl.BlockSpec((L, E), lambda b: (b, 0)),
            pl.BlockSpec((8, E), lambda b: (0, 0)),
        ],
        out_specs=pl.BlockSpec((KP, E), lambda b: (b, 0)),
        out_shape=jax.ShapeDtypeStruct((B * KP, E), F32),
    )(rank_row, xpn, wts)

    G, VT2, MR, A_tc = _ssm_tables(A, B_param, C_param, TC)
    xproc = pl.pallas_call(
        functools.partial(_k45_body, KP=KP, TC=TC, E=E, S=S),
        grid=(B,),
        in_specs=[
            pl.BlockSpec((KP, E), lambda b: (b, 0)),
            pl.BlockSpec((TC, E), lambda b: (0, 0)),
            pl.BlockSpec((TC, S, E), lambda b: (0, 0, 0)),
            pl.BlockSpec((S, TC), lambda b: (0, 0)),
            pl.BlockSpec((S, S), lambda b: (0, 0)),
            pl.BlockSpec((E, C), lambda b: (0, 0)),
            pl.BlockSpec((1, C), lambda b: (0, 0)),
        ],
        out_specs=pl.BlockSpec((KP, C), lambda b: (b, 0)),
        out_shape=jax.ShapeDtypeStruct((B * KP, C), BF16),
    )(xc, G, VT2, MR, A_tc, W_out.astype(BF16), bout2)

    TLS = L // 8
    out = pl.pallas_call(
        functools.partial(_k6_body, KP=KP, K=K),
        grid=(B, 8),
        in_specs=[
            pl.BlockSpec((1, TLS, 1), lambda b, t: (b, t, 0)),
            pl.BlockSpec((KP, C), lambda b, t: (b, 0)),
            pl.BlockSpec((TLS, C), lambda b, t: (b * 8 + t, 0)),
        ],
        out_specs=pl.BlockSpec((TLS, C), lambda b, t: (b * 8 + t, 0)),
        out_shape=jax.ShapeDtypeStruct((N, C), F32),
    )(rank, xproc, xf)

    return out.reshape(B, L, C)


# speedup vs baseline: 4.9983x; 1.0866x over previous
import functools

import jax
import jax.numpy as jnp
from jax.experimental import pallas as pl

F32 = jnp.float32
BF16 = jnp.bfloat16


def _k1_body(x_ref, win_ref, bin_ref, wq_ref, bq_ref, wk_ref, bk_ref,
             alpha_ref, nw_ref, nb_ref, xpn_ref, q_ref, k_ref):
    x = x_ref[...]
    xn = jnp.tanh(alpha_ref[0, 0] * x) * nw_ref[...] + nb_ref[...]
    xp = jnp.dot(xn.astype(BF16), win_ref[...],
                 preferred_element_type=F32) + bin_ref[...]
    ss = jnp.sum(xp * xp, axis=1, keepdims=True)
    nrm = jnp.maximum(jnp.sqrt(ss), 1e-12)
    xpn = xp / nrm
    xpn_ref[...] = xpn.astype(BF16)
    q_ref[...] = jnp.dot(xpn, wq_ref[...], preferred_element_type=F32) + bq_ref[0, 0]
    k_ref[...] = jnp.dot(xpn, wk_ref[...], preferred_element_type=F32) + bk_ref[0, 0]


def _k2a_body(qcol_ref, krow_ref, imp_ref, *, L, CH):
    krow = krow_ref[0]
    acc = jnp.zeros((1, L), F32)
    for c in range(L // CH):
        qc = qcol_ref[0, c * CH:(c + 1) * CH, :]
        s = qc * krow
        m = jnp.max(s, axis=1, keepdims=True)
        e = jnp.exp(s - m)
        z = jnp.sum(e, axis=1, keepdims=True)
        acc = acc + jnp.sum(e / z, axis=0, keepdims=True)
    imp_ref[0] = acc * (1.0 / L)


def _k2b_body(imp_row_ref, imp_col_ref, rank_ref, *, L, CH):
    row = imp_row_ref[0]
    for c in range(L // CH):
        col = imp_col_ref[0, c * CH:(c + 1) * CH, :]
        jidx = jax.lax.broadcasted_iota(jnp.int32, (CH, L), 1)
        iidx = jax.lax.broadcasted_iota(jnp.int32, (CH, L), 0) + c * CH
        gt = (row > col).astype(jnp.int32)
        eq = ((row == col) & (jidx < iidx)).astype(jnp.int32)
        rank = jnp.sum(gt + eq, axis=1, keepdims=True)
        rank_ref[0, c * CH:(c + 1) * CH, :] = rank


def _k345_body(rank_ref, xpn_ref, wts_ref, g_ref, vt2_ref, mr_ref, apow_ref,
               wout_ref, bout_ref, xproc_ref, *, KP, TC, K, E, S):
    rank = rank_ref[0]
    r_iota = jax.lax.broadcasted_iota(jnp.int32, (KP, rank.shape[1]), 0)
    p = jnp.where((rank == r_iota) & (r_iota < K), 1.0, 0.0).astype(BF16)
    sf = jnp.dot(p, xpn_ref[...], preferred_element_type=F32)
    xc_all = wts_ref[3:4, :] * sf
    for j in range(3):
        shift = 3 - j
        sh = jnp.concatenate(
            [jnp.zeros((shift, E), F32), sf[:KP - shift, :]], axis=0)
        xc_all = xc_all + wts_ref[j:j + 1, :] * sh
    h = jnp.zeros((S, E), F32)
    g = g_ref[...]
    vt2 = vt2_ref[...]
    mr = mr_ref[...]
    apow = apow_ref[...]
    wout = wout_ref[...]
    for c in range(KP // TC):
        xc = xc_all[c * TC:(c + 1) * TC, :]
        carry = jnp.sum(vt2 * h[None, :, :], axis=1)
        xp = jnp.concatenate([jnp.zeros((TC - 1, E), F32), xc], axis=0)
        intra = g[0:1, :] * xc
        for d in range(1, TC):
            intra = intra + g[d:d + 1, :] * xp[TC - 1 - d:2 * TC - 1 - d, :]
        xssm = intra + carry
        xproc_ref[c * TC:(c + 1) * TC, :] = (
            jnp.dot(xssm.astype(BF16), wout, preferred_element_type=F32)
            + bout_ref[...]).astype(BF16)
        h = (jnp.dot(apow, h, preferred_element_type=F32)
             + jnp.dot(mr, xc, preferred_element_type=F32))


def _k6_body(rank_ref, xproc_ref, x_ref, out_ref, *, KP, K):
    rcol = rank_ref[0]
    r_iota = jax.lax.broadcasted_iota(jnp.int32, (rcol.shape[0], KP), 1)
    pt = jnp.where((rcol == r_iota) & (r_iota < K), 1.0, 0.0).astype(BF16)
    out_ref[...] = (jnp.dot(pt, xproc_ref[...], preferred_element_type=F32)
                    + x_ref[...])


def _ssm_tables(A, B_param, C_param, TC):
    S = A.shape[0]
    bsig = jax.nn.sigmoid(B_param).reshape(1, S)
    csigT = jax.nn.sigmoid(C_param).T
    AT = A.T
    U = csigT[None]
    m = bsig
    P = AT
    n = 1
    while n < TC:
        U = jnp.concatenate([U, jnp.einsum('sk,dkm->dsm', P, U)], axis=0)
        m = jnp.concatenate([m, m @ P], axis=0)
        P = P @ P
        n *= 2
    U = U[:TC]
    m = m[:TC]
    G = jnp.einsum('os,dsm->dm', bsig, U)
    VT2 = jnp.einsum('sk,dkm->dsm', AT, U)
    MR = m[::-1].T
    A_tc = P.T
    return G, VT2, MR, A_tc


def kernel(x, norm_alpha, norm_weight, norm_bias, W_in, b_in, W_out, b_out,
           A, B_param, C_param, conv_w, W_q, b_q, W_k, b_k):
    B, L, C = x.shape
    E = W_in.shape[1]
    S = A.shape[0]
    K = max(1, int(L * 0.3))
    TC = 32
    KP = ((K + TC - 1) // TC) * TC
    RT = 256
    CH = 256
    N = B * L

    xf = x.reshape(N, C)
    alpha2 = norm_alpha.reshape(1, 1)
    nw2 = norm_weight.reshape(1, C)
    nb2 = norm_bias.reshape(1, C)
    bin2 = b_in.reshape(1, E)
    bout2 = b_out.reshape(1, C)
    bq2 = b_q.reshape(1, 1)
    bk2 = b_k.reshape(1, 1)
    wts = jnp.concatenate([conv_w[:, 0, :].T, jnp.zeros((4, E), F32)], axis=0)

    xpn, q, k = pl.pallas_call(
        _k1_body,
        grid=(N // RT,),
        in_specs=[
            pl.BlockSpec((RT, C), lambda i: (i, 0)),
            pl.BlockSpec((C, E), lambda i: (0, 0)),
            pl.BlockSpec((1, E), lambda i: (0, 0)),
            pl.BlockSpec((E, 1), lambda i: (0, 0)),
            pl.BlockSpec((1, 1), lambda i: (0, 0)),
            pl.BlockSpec((E, 1), lambda i: (0, 0)),
            pl.BlockSpec((1, 1), lambda i: (0, 0)),
            pl.BlockSpec((1, 1), lambda i: (0, 0)),
            pl.BlockSpec((1, C), lambda i: (0, 0)),
            pl.BlockSpec((1, C), lambda i: (0, 0)),
        ],
        out_specs=[
            pl.BlockSpec((RT, E), lambda i: (i, 0)),
            pl.BlockSpec((RT, 1), lambda i: (i, 0)),
            pl.BlockSpec((RT, 1), lambda i: (i, 0)),
        ],
        out_shape=[
            jax.ShapeDtypeStruct((N, E), BF16),
            jax.ShapeDtypeStruct((N, 1), F32),
            jax.ShapeDtypeStruct((N, 1), F32),
        ],
    )(xf, W_in.astype(BF16), bin2, W_q, bq2, W_k, bk2, alpha2, nw2, nb2)

    qcol = q.reshape(B, L, 1)
    krow = k.reshape(B, 1, L)

    imp = pl.pallas_call(
        functools.partial(_k2a_body, L=L, CH=CH),
        grid=(B,),
        in_specs=[
            pl.BlockSpec((1, L, 1), lambda b: (b, 0, 0)),
            pl.BlockSpec((1, 1, L), lambda b: (b, 0, 0)),
        ],
        out_specs=pl.BlockSpec((1, 1, L), lambda b: (b, 0, 0)),
        out_shape=jax.ShapeDtypeStruct((B, 1, L), F32),
    )(qcol, krow)

    imp_col = jnp.swapaxes(imp, 1, 2)

    rank = pl.pallas_call(
        functools.partial(_k2b_body, L=L, CH=CH),
        grid=(B,),
        in_specs=[
            pl.BlockSpec((1, 1, L), lambda b: (b, 0, 0)),
            pl.BlockSpec((1, L, 1), lambda b: (b, 0, 0)),
        ],
        out_specs=pl.BlockSpec((1, L, 1), lambda b: (b, 0, 0)),
        out_shape=jax.ShapeDtypeStruct((B, L, 1), jnp.int32),
    )(imp, imp_col)

    rank_row = jnp.swapaxes(rank, 1, 2)

    G, VT2, MR, A_tc = _ssm_tables(A, B_param, C_param, TC)
    xproc = pl.pallas_call(
        functools.partial(_k345_body, KP=KP, TC=TC, K=K, E=E, S=S),
        grid=(B,),
        in_specs=[
            pl.BlockSpec((1, 1, L), lambda b: (b, 0, 0)),
            pl.BlockSpec((L, E), lambda b: (b, 0)),
            pl.BlockSpec((8, E), lambda b: (0, 0)),
            pl.BlockSpec((TC, E), lambda b: (0, 0)),
            pl.BlockSpec((TC, S, E), lambda b: (0, 0, 0)),
            pl.BlockSpec((S, TC), lambda b: (0, 0)),
            pl.BlockSpec((S, S), lambda b: (0, 0)),
            pl.BlockSpec((E, C), lambda b: (0, 0)),
            pl.BlockSpec((1, C), lambda b: (0, 0)),
        ],
        out_specs=pl.BlockSpec((KP, C), lambda b: (b, 0)),
        out_shape=jax.ShapeDtypeStruct((B * KP, C), BF16),
    )(rank_row, xpn, wts, G, VT2, MR, A_tc, W_out.astype(BF16), bout2)

    TLS = L // 8
    out = pl.pallas_call(
        functools.partial(_k6_body, KP=KP, K=K),
        grid=(B, 8),
        in_specs=[
            pl.BlockSpec((1, TLS, 1), lambda b, t: (b, t, 0)),
            pl.BlockSpec((KP, C), lambda b, t: (b, 0)),
            pl.BlockSpec((TLS, C), lambda b, t: (b * 8 + t, 0)),
        ],
        out_specs=pl.BlockSpec((TLS, C), lambda b, t: (b * 8 + t, 0)),
        out_shape=jax.ShapeDtypeStruct((N, C), F32),
    )(rank, xproc, xf)

    return out.reshape(B, L, C)
